# Initial kernel scaffold; baseline (speedup 1.0000x reference)
import jax
import jax.numpy as jnp
from jax.experimental import pallas as pl


def kernel(x, weight):
    raise NotImplementedError("write your pallas kernel here")


# speedup vs baseline: 1.7968x; 1.7968x over previous
import functools

import jax
import jax.numpy as jnp
from jax import lax
from jax.experimental import pallas as pl
from jax.experimental.pallas import tpu as pltpu
from jax.experimental.pallas import tpu_sc as plsc

_D = 64
_IW = 128
_JROWS = 4
_CHUNK = _IW * _JROWS


def _make_gather(n_rows: int, d: int):
    info = plsc.get_sparse_core_info()
    nw = info.num_cores * info.num_subcores
    nc = info.num_cores
    rows_per_w = n_rows // nw
    idx_rows_per_w = rows_per_w // _IW
    chunks = idx_rows_per_w // _JROWS
    assert rows_per_w * nw == n_rows and chunks * _JROWS == idx_rows_per_w

    mesh = plsc.VectorSubcoreMesh(core_axis_name="c", subcore_axis_name="s")

    @functools.partial(
        pl.kernel,
        mesh=mesh,
        compiler_params=pltpu.CompilerParams(use_tc_tiling_on_sc=False),
        out_type=jax.ShapeDtypeStruct((n_rows, d), jnp.float32),
        scratch_types=[
            pltpu.VMEM((_JROWS, _IW), jnp.int32),
            pltpu.VMEM((_CHUNK, d), jnp.float32),
            pltpu.SemaphoreType.DMA,
        ],
    )
    def k(table_hbm, idx_hbm, out_hbm, idx_v, rows_v, sem):
        wid = lax.axis_index("s") * nc + lax.axis_index("c")
        idx_row0 = wid * idx_rows_per_w
        out_row0 = wid * rows_per_w

        def body(g, carry):
            pltpu.sync_copy(idx_hbm.at[pl.ds(idx_row0 + g * _JROWS, _JROWS)],
                            idx_v)
            cps = [
                pltpu.async_copy(table_hbm.at[idx_v.at[j]],
                                 rows_v.at[pl.ds(j * _IW, _IW)], sem)
                for j in range(_JROWS)
            ]
            for cp in cps:
                cp.wait()
            pltpu.sync_copy(rows_v,
                            out_hbm.at[pl.ds(out_row0 + g * _CHUNK, _CHUNK)])
            return carry

        lax.fori_loop(0, chunks, body, 0)

    return k


def kernel(x, weight):
    b, s = x.shape
    n = b * s
    d = weight.shape[1]
    idx2d = x.reshape(n // _IW, _IW).astype(jnp.int32)
    out = _make_gather(n, d)(weight, idx2d)
    return out.reshape(b, s, d)


# speedup vs baseline: 1.8772x; 1.0448x over previous
import functools

import jax
import jax.numpy as jnp
from jax import lax
from jax.experimental import pallas as pl
from jax.experimental.pallas import tpu as pltpu
from jax.experimental.pallas import tpu_sc as plsc

_IW = 128
_JROWS = 4
_CHUNK = _IW * _JROWS


def _make_gather(n_rows: int, d: int):
    info = plsc.get_sparse_core_info()
    nw = info.num_cores * info.num_subcores
    nc = info.num_cores
    rows_per_w = n_rows // nw
    idx_rows_per_w = rows_per_w // _IW
    chunks = idx_rows_per_w // _JROWS
    assert rows_per_w * nw == n_rows and chunks * _JROWS == idx_rows_per_w
    assert chunks % 2 == 0 and chunks >= 4

    mesh = plsc.VectorSubcoreMesh(core_axis_name="c", subcore_axis_name="s")

    @functools.partial(
        pl.kernel,
        mesh=mesh,
        compiler_params=pltpu.CompilerParams(use_tc_tiling_on_sc=False),
        out_type=jax.ShapeDtypeStruct((n_rows, d), jnp.float32),
        scratch_types=[
            pltpu.VMEM((idx_rows_per_w, _IW), jnp.int32),
            pltpu.VMEM((_CHUNK, d), jnp.float32),
            pltpu.VMEM((_CHUNK, d), jnp.float32),
            pltpu.SemaphoreType.DMA,
            pltpu.SemaphoreType.DMA,
        ],
    )
    def k(table_hbm, idx_hbm, out_hbm, idx_v, rows0, rows1, sem0, sem1):
        wid = lax.axis_index("s") * nc + lax.axis_index("c")
        idx_row0 = wid * idx_rows_per_w
        out_row0 = wid * rows_per_w

        pltpu.sync_copy(idx_hbm.at[pl.ds(idx_row0, idx_rows_per_w)], idx_v)

        def fire(c, rows_v, sem):
            for j in range(_JROWS):
                pltpu.async_copy(table_hbm.at[idx_v.at[c * _JROWS + j]],
                                 rows_v.at[pl.ds(j * _IW, _IW)], sem)

        def drain_and_write(c, rows_v, sem):
            for j in range(_JROWS):
                pltpu.make_async_copy(
                    table_hbm.at[idx_v.at[j]],
                    rows_v.at[pl.ds(j * _IW, _IW)], sem).wait()
            pltpu.sync_copy(rows_v,
                            out_hbm.at[pl.ds(out_row0 + c * _CHUNK, _CHUNK)])

        fire(0, rows0, sem0)

        def body(i, carry):
            c = 2 * i
            fire(c + 1, rows1, sem1)
            drain_and_write(c, rows0, sem0)
            fire(c + 2, rows0, sem0)
            drain_and_write(c + 1, rows1, sem1)
            return carry

        lax.fori_loop(0, chunks // 2 - 1, body, 0)

        c = chunks - 2
        fire(c + 1, rows1, sem1)
        drain_and_write(c, rows0, sem0)
        drain_and_write(c + 1, rows1, sem1)

    return k


def kernel(x, weight):
    b, s = x.shape
    n = b * s
    d = weight.shape[1]
    idx2d = x.reshape(n // _IW, _IW).astype(jnp.int32)
    out = _make_gather(n, d)(weight, idx2d)
    return out.reshape(b, s, d)


# speedup vs baseline: 2.0731x; 1.1044x over previous
import functools

import jax
import jax.numpy as jnp
from jax import lax
from jax.experimental import pallas as pl
from jax.experimental.pallas import tpu as pltpu
from jax.experimental.pallas import tpu_sc as plsc

_IW = 128
_JROWS = 4
_CHUNK = _IW * _JROWS


def _make_gather(n_rows: int, n_vocab: int, d: int):
    info = plsc.get_sparse_core_info()
    nw = info.num_cores * info.num_subcores
    nc = info.num_cores
    rows_per_w = n_rows // nw
    idx_rows_per_w = rows_per_w // _IW
    chunks = idx_rows_per_w // _JROWS
    assert rows_per_w * nw == n_rows and chunks * _JROWS == idx_rows_per_w
    assert chunks % 2 == 0 and chunks >= 4

    mesh = plsc.VectorSubcoreMesh(core_axis_name="c", subcore_axis_name="s")

    @functools.partial(
        pl.kernel,
        mesh=mesh,
        compiler_params=pltpu.CompilerParams(use_tc_tiling_on_sc=False),
        out_type=jax.ShapeDtypeStruct((n_rows, d), jnp.float32),
        scratch_types=[
            pltpu.VMEM((idx_rows_per_w, _IW), jnp.int32),
            pltpu.VMEM((_CHUNK, d), jnp.float32),
            pltpu.VMEM((_CHUNK, d), jnp.float32),
            pltpu.SemaphoreType.DMA,
            pltpu.SemaphoreType.DMA,
        ],
    )
    def k(table_hbm, idx_hbm, out128, idx_v, rows0, rows1, sem0, sem1):
        wid = lax.axis_index("s") * nc + lax.axis_index("c")
        idx_row0 = wid * idx_rows_per_w
        out_row0 = wid * rows_per_w

        pltpu.sync_copy(idx_hbm.at[pl.ds(idx_row0, idx_rows_per_w)], idx_v)

        def fire(c, rows_v, sem):
            for j in range(_JROWS):
                pltpu.async_copy(table_hbm.at[idx_v.at[c * _JROWS + j]],
                                 rows_v.at[pl.ds(j * _IW, _IW)], sem)

        def drain_and_write(c, rows_v, sem):
            for j in range(_JROWS):
                pltpu.make_async_copy(
                    table_hbm.at[idx_v.at[j]],
                    rows_v.at[pl.ds(j * _IW, _IW)], sem).wait()
            pltpu.sync_copy(rows_v,
                            out128.at[pl.ds(out_row0 + c * _CHUNK, _CHUNK)])

        fire(0, rows0, sem0)

        def body(i, carry):
            c = 2 * i
            fire(c + 1, rows1, sem1)
            drain_and_write(c, rows0, sem0)
            fire(c + 2, rows0, sem0)
            drain_and_write(c + 1, rows1, sem1)
            return carry

        lax.fori_loop(0, chunks // 2 - 1, body, 0)

        c = chunks - 2
        fire(c + 1, rows1, sem1)
        drain_and_write(c, rows0, sem0)
        drain_and_write(c + 1, rows1, sem1)

    return k


def kernel(x, weight):
    b, s = x.shape
    n = b * s
    v, d = weight.shape
    w2 = jnp.pad(weight, ((0, 0), (0, 128 - d))).reshape(2 * v, d)
    idx2d = (x.T.reshape(n // _IW, _IW) * 2).astype(jnp.int32)
    out128 = _make_gather(n, 2 * v, d)(w2, idx2d)
    return out128.reshape(s, b, d).transpose(1, 0, 2)
